# Initial kernel scaffold; baseline (speedup 1.0000x reference)
import jax
import jax.numpy as jnp
from jax.experimental import pallas as pl


def kernel(x_encoded, register, W, b):
    raise NotImplementedError("write your pallas kernel here")


# speedup vs baseline: 1.3239x; 1.3239x over previous
import functools

import jax
import jax.numpy as jnp
from jax import lax
from jax.experimental import pallas as pl
from jax.experimental.pallas import tpu as pltpu
from jax.experimental.pallas import tpu_sc as plsc

B, K, D, T = 16384, 8192, 64, 3
BB = 256
BK = 1024


def _dist_kernel(x_ref, regt_ref, wt_ref, b_ref, min_ref, idx_ref):
    xe = jnp.dot(x_ref[...].astype(jnp.bfloat16),
                 wt_ref[...].astype(jnp.bfloat16),
                 preferred_element_type=jnp.float32) + b_ref[...]
    x2 = jnp.sum(xe * xe, axis=1, keepdims=True)
    best = None
    bidx = None
    for c in range(K // BK):
        chunk = regt_ref[:, c * BK:(c + 1) * BK]
        c2 = jnp.sum(chunk * chunk, axis=0, keepdims=True)
        prod = jnp.dot(xe.astype(jnp.bfloat16),
                       chunk.astype(jnp.bfloat16),
                       preferred_element_type=jnp.float32)
        d2 = jnp.maximum(x2 + c2 - 2.0 * prod, 0.0)
        tmin = jnp.min(d2, axis=1, keepdims=True)
        iota = lax.broadcasted_iota(jnp.int32, d2.shape, 1) + c * BK
        tidx = jnp.min(jnp.where(d2 == tmin, iota, K), axis=1,
                       keepdims=True)
        if best is None:
            best, bidx = tmin, tidx
        else:
            upd = tmin < best
            bidx = jnp.where(upd, tidx, bidx)
            best = jnp.where(upd, tmin, best)
    min_ref[...] = jnp.sqrt(best)
    idx_ref[...] = bidx


def _distance_argmin(x_encoded, regt, wt, b2d):
    grid = (B // BB,)
    return pl.pallas_call(
        _dist_kernel,
        grid=grid,
        in_specs=[
            pl.BlockSpec((BB, D), lambda i: (i, 0)),
            pl.BlockSpec((D, K), lambda i: (0, 0)),
            pl.BlockSpec((D, D), lambda i: (0, 0)),
            pl.BlockSpec((1, D), lambda i: (0, 0)),
        ],
        out_specs=[
            pl.BlockSpec((BB, 1), lambda i: (i, 0)),
            pl.BlockSpec((BB, 1), lambda i: (i, 0)),
        ],
        out_shape=[
            jax.ShapeDtypeStruct((B, 1), jnp.float32),
            jax.ShapeDtypeStruct((B, 1), jnp.int32),
        ],
        compiler_params=pltpu.CompilerParams(
            dimension_semantics=("arbitrary",),
        ),
    )(x_encoded, regt, wt, b2d)


def _make_sc_gather():
    info = plsc.get_sparse_core_info()
    nw = info.num_cores * info.num_subcores
    b_per_w = B // nw
    mesh = plsc.VectorSubcoreMesh(core_axis_name="c", subcore_axis_name="s")

    @functools.partial(
        pl.kernel, mesh=mesh,
        out_type=jax.ShapeDtypeStruct((B, D), jnp.float32),
        compiler_params=pltpu.CompilerParams(use_tc_tiling_on_sc=False),
        scratch_types=[
            pltpu.VMEM((b_per_w,), jnp.int32),
            pltpu.VMEM((b_per_w, D), jnp.float32),
            pltpu.SemaphoreType.DMA,
        ],
    )
    def gather(table_hbm, idx_hbm, out_hbm, idx_v, rows_v, sem):
        wid = lax.axis_index("s") * info.num_cores + lax.axis_index("c")
        base = wid * b_per_w
        pltpu.sync_copy(idx_hbm.at[pl.ds(base, b_per_w)], idx_v)
        pltpu.async_copy(table_hbm.at[idx_v], rows_v, sem).wait()
        pltpu.sync_copy(rows_v, out_hbm.at[pl.ds(base, b_per_w)])

    return gather


_sc_gather = None


def kernel(x_encoded, register, W, b):
    global _sc_gather
    if _sc_gather is None:
        _sc_gather = _make_sc_gather()
    regt = register.T
    wt = W.T
    b2d = b.reshape(1, D)
    min_d, idx2d = _distance_argmin(x_encoded, regt, wt, b2d)
    closest_idx = idx2d.reshape(B)
    min_dist = min_d.reshape(B)
    selected = _sc_gather(register, closest_idx)
    register_tokens = jnp.broadcast_to(selected[:, None, :], (B, T, D))
    return register_tokens, closest_idx, min_dist


# speedup vs baseline: 1.5964x; 1.2059x over previous
import functools

import jax
import jax.numpy as jnp
from jax import lax
from jax.experimental import pallas as pl
from jax.experimental.pallas import tpu as pltpu
from jax.experimental.pallas import tpu_sc as plsc

B, K, D, T = 16384, 8192, 64, 3
BB = 256
BK = 1024


def _dist_kernel(x_ref, regt_ref, wt_ref, b_ref, min_ref, idx_ref,
                 c2_ref, rbf_ref):
    @pl.when(pl.program_id(0) == 0)
    def _init():
        for c in range(K // BK):
            ch = regt_ref[:, c * BK:(c + 1) * BK]
            c2_ref[:, c * BK:(c + 1) * BK] = jnp.sum(ch * ch, axis=0,
                                                     keepdims=True)
            rbf_ref[:, c * BK:(c + 1) * BK] = ch.astype(jnp.bfloat16)

    xe = jnp.dot(x_ref[...].astype(jnp.bfloat16),
                 wt_ref[...].astype(jnp.bfloat16),
                 preferred_element_type=jnp.float32) + b_ref[...]
    x2 = jnp.sum(xe * xe, axis=1, keepdims=True)
    xm2_bf = ((-2.0) * xe).astype(jnp.bfloat16)
    iota = lax.broadcasted_iota(jnp.int32, (BB, BK), 1)
    best = None
    bidx = None
    for c in range(K // BK):
        pm2 = jnp.dot(xm2_bf, rbf_ref[:, c * BK:(c + 1) * BK],
                      preferred_element_type=jnp.float32)
        d2n = c2_ref[:, c * BK:(c + 1) * BK] + pm2
        tmin = jnp.min(d2n, axis=1, keepdims=True)
        tloc = jnp.min(jnp.where(d2n == tmin, iota, K), axis=1,
                       keepdims=True)
        tidx = tloc + c * BK
        if best is None:
            best, bidx = tmin, tidx
        else:
            upd = tmin < best
            bidx = jnp.where(upd, tidx, bidx)
            best = jnp.where(upd, tmin, best)
    min_ref[...] = jnp.sqrt(jnp.maximum(best + x2, 0.0))
    idx_ref[...] = bidx


def _distance_argmin(x_encoded, regt, wt, b2d):
    grid = (B // BB,)
    return pl.pallas_call(
        _dist_kernel,
        grid=grid,
        in_specs=[
            pl.BlockSpec((BB, D), lambda i: (i, 0)),
            pl.BlockSpec((D, K), lambda i: (0, 0)),
            pl.BlockSpec((D, D), lambda i: (0, 0)),
            pl.BlockSpec((1, D), lambda i: (0, 0)),
        ],
        out_specs=[
            pl.BlockSpec((BB, 1), lambda i: (i, 0)),
            pl.BlockSpec((BB, 1), lambda i: (i, 0)),
        ],
        out_shape=[
            jax.ShapeDtypeStruct((B, 1), jnp.float32),
            jax.ShapeDtypeStruct((B, 1), jnp.int32),
        ],
        scratch_shapes=[
            pltpu.VMEM((1, K), jnp.float32),
            pltpu.VMEM((D, K), jnp.bfloat16),
        ],
        compiler_params=pltpu.CompilerParams(
            dimension_semantics=("arbitrary",),
        ),
    )(x_encoded, regt, wt, b2d)


def _make_sc_gather():
    info = plsc.get_sparse_core_info()
    nw = info.num_cores * info.num_subcores
    b_per_w = B // nw
    mesh = plsc.VectorSubcoreMesh(core_axis_name="c", subcore_axis_name="s")

    @functools.partial(
        pl.kernel, mesh=mesh,
        out_type=jax.ShapeDtypeStruct((B, D), jnp.float32),
        compiler_params=pltpu.CompilerParams(use_tc_tiling_on_sc=False),
        scratch_types=[
            pltpu.VMEM((b_per_w,), jnp.int32),
            pltpu.VMEM((b_per_w, D), jnp.float32),
            pltpu.SemaphoreType.DMA,
        ],
    )
    def gather(table_hbm, idx_hbm, out_hbm, idx_v, rows_v, sem):
        wid = lax.axis_index("s") * info.num_cores + lax.axis_index("c")
        base = wid * b_per_w
        pltpu.sync_copy(idx_hbm.at[pl.ds(base, b_per_w)], idx_v)
        pltpu.async_copy(table_hbm.at[idx_v], rows_v, sem).wait()
        pltpu.sync_copy(rows_v, out_hbm.at[pl.ds(base, b_per_w)])

    return gather


_sc_gather = None


def kernel(x_encoded, register, W, b):
    global _sc_gather
    if _sc_gather is None:
        _sc_gather = _make_sc_gather()
    regt = register.T
    wt = W.T
    b2d = b.reshape(1, D)
    min_d, idx2d = _distance_argmin(x_encoded, regt, wt, b2d)
    closest_idx = idx2d.reshape(B)
    min_dist = min_d.reshape(B)
    selected = _sc_gather(register, closest_idx)
    register_tokens = jnp.broadcast_to(selected[:, None, :], (B, T, D))
    return register_tokens, closest_idx, min_dist


# speedup vs baseline: 2.1119x; 1.3229x over previous
import functools

import jax
import jax.numpy as jnp
from jax import lax
from jax.experimental import pallas as pl
from jax.experimental.pallas import tpu as pltpu
from jax.experimental.pallas import tpu_sc as plsc

B, K, D, T = 16384, 8192, 64, 3
BB = 1024
BK = 1024


def _dist_kernel(x_ref, regt_ref, wt_ref, b_ref, min_ref, idx_ref,
                 c2_ref, rbf_ref):
    @pl.when(pl.program_id(0) == 0)
    def _init():
        for c in range(K // BK):
            ch = regt_ref[:, c * BK:(c + 1) * BK]
            c2_ref[:, c * BK:(c + 1) * BK] = jnp.sum(ch * ch, axis=0,
                                                     keepdims=True)
            rbf_ref[:, c * BK:(c + 1) * BK] = ch.astype(jnp.bfloat16)

    xe = jnp.dot(x_ref[...].astype(jnp.bfloat16),
                 wt_ref[...].astype(jnp.bfloat16),
                 preferred_element_type=jnp.float32) + b_ref[...]
    x2 = jnp.sum(xe * xe, axis=1, keepdims=True)
    xm2_bf = ((-2.0) * xe).astype(jnp.bfloat16)
    iota = lax.broadcasted_iota(jnp.int32, (1, BK), 1).astype(jnp.float32)
    best = None
    bidx = None
    for c in range(K // BK):
        pm2 = jnp.dot(xm2_bf, rbf_ref[:, c * BK:(c + 1) * BK],
                      preferred_element_type=jnp.float32)
        d2n = c2_ref[:, c * BK:(c + 1) * BK] + pm2
        tmin = jnp.min(d2n, axis=1, keepdims=True)
        tloc = jnp.min(jnp.where(d2n == tmin, iota, jnp.float32(K)),
                       axis=1, keepdims=True)
        tidx = tloc + jnp.float32(c * BK)
        if best is None:
            best, bidx = tmin, tidx
        else:
            upd = tmin < best
            bidx = jnp.where(upd, tidx, bidx)
            best = jnp.where(upd, tmin, best)
    min_ref[...] = jnp.sqrt(jnp.maximum(best + x2, 0.0))
    idx_ref[...] = bidx.astype(jnp.int32)


def _distance_argmin(x_encoded, regt, wt, b2d):
    grid = (B // BB,)
    return pl.pallas_call(
        _dist_kernel,
        grid=grid,
        in_specs=[
            pl.BlockSpec((BB, D), lambda i: (i, 0)),
            pl.BlockSpec((D, K), lambda i: (0, 0)),
            pl.BlockSpec((D, D), lambda i: (0, 0)),
            pl.BlockSpec((1, D), lambda i: (0, 0)),
        ],
        out_specs=[
            pl.BlockSpec((BB, 1), lambda i: (i, 0)),
            pl.BlockSpec((BB, 1), lambda i: (i, 0)),
        ],
        out_shape=[
            jax.ShapeDtypeStruct((B, 1), jnp.float32),
            jax.ShapeDtypeStruct((B, 1), jnp.int32),
        ],
        scratch_shapes=[
            pltpu.VMEM((1, K), jnp.float32),
            pltpu.VMEM((D, K), jnp.bfloat16),
        ],
        compiler_params=pltpu.CompilerParams(
            dimension_semantics=("arbitrary",),
        ),
    )(x_encoded, regt, wt, b2d)


def _make_sc_gather():
    info = plsc.get_sparse_core_info()
    nw = info.num_cores * info.num_subcores
    b_per_w = B // nw
    mesh = plsc.VectorSubcoreMesh(core_axis_name="c", subcore_axis_name="s")

    @functools.partial(
        pl.kernel, mesh=mesh,
        out_type=jax.ShapeDtypeStruct((B, D), jnp.float32),
        compiler_params=pltpu.CompilerParams(use_tc_tiling_on_sc=False),
        scratch_types=[
            pltpu.VMEM((b_per_w,), jnp.int32),
            pltpu.VMEM((b_per_w, D), jnp.float32),
            pltpu.SemaphoreType.DMA,
        ],
    )
    def gather(table_hbm, idx_hbm, out_hbm, idx_v, rows_v, sem):
        wid = lax.axis_index("s") * info.num_cores + lax.axis_index("c")
        base = wid * b_per_w
        pltpu.sync_copy(idx_hbm.at[pl.ds(base, b_per_w)], idx_v)
        pltpu.async_copy(table_hbm.at[idx_v], rows_v, sem).wait()
        pltpu.sync_copy(rows_v, out_hbm.at[pl.ds(base, b_per_w)])

    return gather


_sc_gather = None


def kernel(x_encoded, register, W, b):
    global _sc_gather
    if _sc_gather is None:
        _sc_gather = _make_sc_gather()
    regt = register.T
    wt = W.T
    b2d = b.reshape(1, D)
    min_d, idx2d = _distance_argmin(x_encoded, regt, wt, b2d)
    closest_idx = idx2d.reshape(B)
    min_dist = min_d.reshape(B)
    selected = _sc_gather(register, closest_idx)
    register_tokens = jnp.broadcast_to(selected[:, None, :], (B, T, D))
    return register_tokens, closest_idx, min_dist
